# baseline (speedup 1.0000x reference)
import jax
import jax.numpy as jnp
from jax.experimental import pallas as pl
from jax.experimental.pallas import tpu as pltpu

_N_CHUNKS = 8


def _copy_body(x_ref, o_ref, sems):
    rows = x_ref.shape[0] // _N_CHUNKS
    for i in range(_N_CHUNKS):
        sl = pl.ds(i * rows, rows)
        pltpu.make_async_copy(x_ref.at[sl], o_ref.at[sl], sems.at[i]).start()
    for i in range(_N_CHUNKS):
        rows_sl = pl.ds(i * rows, rows)
        pltpu.make_async_copy(x_ref.at[rows_sl], o_ref.at[rows_sl], sems.at[i]).wait()


def kernel(x, single_eval_pos):
    return pl.pallas_call(
        _copy_body,
        out_shape=jax.ShapeDtypeStruct(x.shape, x.dtype),
        in_specs=[pl.BlockSpec(memory_space=pltpu.MemorySpace.HBM)],
        out_specs=pl.BlockSpec(memory_space=pltpu.MemorySpace.HBM),
        scratch_shapes=[pltpu.SemaphoreType.DMA((_N_CHUNKS,))],
    )(x)


# speedup vs baseline: 45.7161x; 45.7161x over previous
import jax
import jax.numpy as jnp
from jax.experimental import pallas as pl
from jax.experimental.pallas import tpu as pltpu

_BLOCK_T = 256


def _copy_body(x_ref, o_ref):
    o_ref[...] = x_ref[...]


def kernel(x, single_eval_pos):
    t = x.shape[0]
    return pl.pallas_call(
        _copy_body,
        out_shape=jax.ShapeDtypeStruct(x.shape, x.dtype),
        grid=(t // _BLOCK_T,),
        in_specs=[pl.BlockSpec((_BLOCK_T,) + x.shape[1:], lambda i: (i, 0, 0))],
        out_specs=pl.BlockSpec((_BLOCK_T,) + x.shape[1:], lambda i: (i, 0, 0)),
    )(x)
